# baseline (speedup 1.0000x reference)
import functools

import jax
import jax.numpy as jnp
from jax import lax
from jax.experimental import pallas as pl
from jax.experimental.pallas import tpu as pltpu
from jax.experimental.pallas import tpu_sc as plsc


def _identity_mask_sc(batch_size, dec_seqlen, enc_seqlen):
    info = plsc.get_sparse_core_info()
    nc, ns, lanes = info.num_cores, info.num_subcores, info.num_lanes
    nw = nc * ns

    total_rows = batch_size * dec_seqlen
    rows_per_w = total_rows // nw
    blk_rows = 8
    nblk = rows_per_w // blk_rows
    stride = enc_seqlen + 1
    base_shift = dec_seqlen - blk_rows
    win = blk_rows * enc_seqlen
    src_len = base_shift + win + lanes
    subc_per_batch = dec_seqlen // rows_per_w

    mesh = plsc.VectorSubcoreMesh(core_axis_name="c", subcore_axis_name="s")

    @functools.partial(
        pl.kernel,
        mesh=mesh,
        out_type=jax.ShapeDtypeStruct((total_rows * enc_seqlen,), jnp.float32),
        scratch_types=[
            pltpu.VMEM((src_len,), jnp.float32),
            pltpu.VMEM((lanes,), jnp.float32),
            pltpu.SemaphoreType.DMA,
            pltpu.SemaphoreType.DMA,
        ],
    )
    def k(zeros_hbm, val_hbm, out_hbm, stair_v, val_v, sem, sem2):
        wid = lax.axis_index("s") * nc + lax.axis_index("c")
        pltpu.sync_copy(zeros_hbm, stair_v)
        pltpu.sync_copy(val_hbm, val_v)
        val = val_v[...]
        lane_iota = lax.iota(jnp.int32, lanes)
        for m in range(blk_rows):
            p = base_shift + m * stride
            base = (p // lanes) * lanes
            stair_v[pl.ds(base, lanes)] = jnp.where(
                lane_iota == (p - base), val, jnp.float32(0.0)
            )
        base_row = wid * rows_per_w
        d0_base = (wid % subc_per_batch) * rows_per_w
        copies = []
        for it in range(nblk):
            r0 = base_row + it * blk_rows
            d0 = d0_base + it * blk_rows
            s = base_shift - d0
            copies.append(
                pltpu.make_async_copy(
                    stair_v.at[pl.ds(s, win)],
                    out_hbm.at[pl.ds(r0 * enc_seqlen, win)],
                    sem if it % 2 == 0 else sem2,
                )
            )
        for c in copies:
            c.start()
        for c in copies:
            c.wait()

    return k, src_len, lanes


def kernel(decoder_states, encoder_states, step):
    batch_size, enc_seqlen, _ = encoder_states.shape
    _, dec_seqlen, _ = decoder_states.shape
    k, src_len, lanes = _identity_mask_sc(batch_size, dec_seqlen, enc_seqlen)
    val = jnp.full((lanes,), -jnp.asarray(step, jnp.int32), dtype=jnp.float32)
    zeros = jnp.zeros((src_len,), dtype=jnp.float32)
    flat = k(zeros, val)
    return flat.reshape(batch_size, dec_seqlen, enc_seqlen)


# speedup vs baseline: 1.0027x; 1.0027x over previous
import functools

import jax
import jax.numpy as jnp
from jax import lax
from jax.experimental import pallas as pl
from jax.experimental.pallas import tpu as pltpu
from jax.experimental.pallas import tpu_sc as plsc


def _identity_mask_sc(batch_size, dec_seqlen, enc_seqlen):
    info = plsc.get_sparse_core_info()
    nc, ns, lanes = info.num_cores, info.num_subcores, info.num_lanes
    nw = nc * ns

    total_rows = batch_size * dec_seqlen
    rows_per_w = total_rows // nw
    blk_rows = 8
    nblk = rows_per_w // blk_rows
    stride = enc_seqlen + 1
    base_shift = dec_seqlen - blk_rows
    win = blk_rows * enc_seqlen
    src_len = base_shift + win + lanes
    subc_per_batch = dec_seqlen // rows_per_w

    mesh = plsc.VectorSubcoreMesh(core_axis_name="c", subcore_axis_name="s")

    @functools.partial(
        pl.kernel,
        mesh=mesh,
        out_type=jax.ShapeDtypeStruct((total_rows * enc_seqlen,), jnp.float32),
        scratch_types=[
            pltpu.VMEM((src_len,), jnp.float32),
            pltpu.VMEM((lanes,), jnp.float32),
            pltpu.SemaphoreType.DMA,
        ],
    )
    def k(zeros_hbm, val_hbm, out_hbm, stair_v, val_v, sem):
        wid = lax.axis_index("s") * nc + lax.axis_index("c")
        pltpu.sync_copy(zeros_hbm, stair_v)
        pltpu.sync_copy(val_hbm, val_v)
        val = val_v[...]
        lane_iota = lax.iota(jnp.int32, lanes)
        for m in range(blk_rows):
            p = base_shift + m * stride
            base = (p // lanes) * lanes
            stair_v[pl.ds(base, lanes)] = jnp.where(
                lane_iota == (p - base), val, jnp.float32(0.0)
            )
        base_row = wid * rows_per_w
        d0_base = (wid % subc_per_batch) * rows_per_w
        copies = []
        for it in range(nblk):
            r0 = base_row + it * blk_rows
            d0 = d0_base + it * blk_rows
            s = base_shift - d0
            copies.append(
                pltpu.make_async_copy(
                    stair_v.at[pl.ds(s, win)],
                    out_hbm.at[pl.ds(r0 * enc_seqlen, win)],
                    sem,
                )
            )
        for c in copies:
            c.start()
        for c in copies:
            c.wait()

    return k, src_len, lanes


def kernel(decoder_states, encoder_states, step):
    batch_size, enc_seqlen, _ = encoder_states.shape
    _, dec_seqlen, _ = decoder_states.shape
    k, src_len, lanes = _identity_mask_sc(batch_size, dec_seqlen, enc_seqlen)
    val = jnp.full((lanes,), -jnp.asarray(step, jnp.int32), dtype=jnp.float32)
    zeros = jnp.zeros((src_len,), dtype=jnp.float32)
    flat = k(zeros, val)
    return flat.reshape(batch_size, dec_seqlen, enc_seqlen)
